# baseline (speedup 1.0000x reference)
import functools

import numpy as np
import jax
import jax.numpy as jnp
from jax import lax
from jax.experimental import pallas as pl
from jax.experimental.pallas import tpu as pltpu
from jax.experimental.pallas import tpu_sc as plsc

N_ITEMS = 10000
D = 128
DQ = D // 2
H = 256
E = 320000
B = 1024
T = 50

_NC = 2
_NS = 16
_NW = _NC * _NS
_C = 128
_HC = _C // 2
_CPW = 80
_JB = 8
_NBLK = _CPW // _JB
_EP = _NW * _CPW * _C
_RPT = 640
_TH = T // 2
_GW = 7
_GP = _NW * _GW * _C
_VP = 10240

_PERM = np.array(
    [32 * (p // 32) + (2 * (p % 32) if p % 32 < 16 else 2 * (p % 32 - 16) + 1)
     for p in range(D)], dtype=np.int32)

_SC_MESH = plsc.VectorSubcoreMesh(core_axis_name="c", subcore_axis_name="s")


def _as_i32_pairs(xb):
    return jax.lax.bitcast_convert_type(
        xb.reshape(xb.shape[0], DQ, 2), jnp.int32)


@functools.partial(
    pl.kernel,
    out_type=[jax.ShapeDtypeStruct((N_ITEMS, D), jnp.float32),
              jax.ShapeDtypeStruct((N_ITEMS, D), jnp.float32)],
    mesh=_SC_MESH,
    scratch_types=[
        pltpu.VMEM((2, _JB, _C), jnp.int32),
        pltpu.VMEM((2, _JB * 2, _HC), jnp.int32),
        pltpu.VMEM((2 * _JB * _C,), jnp.float32),
        pltpu.VMEM((2, _C, DQ), jnp.int32),
        pltpu.VMEM((_HC, D), jnp.float32),
        pltpu.VMEM((_HC, D), jnp.float32),
        pltpu.VMEM_SHARED((N_ITEMS, D), jnp.float32),
        pltpu.SemaphoreType.DMA,
        pltpu.SemaphoreType.DMA,
        pltpu.SemaphoreType.DMA,
        pltpu.SemaphoreType.DMA,
        pltpu.SemaphoreType.DMA,
    ],
    compiler_params=pltpu.CompilerParams(needs_layout_passes=False,
                                         use_tc_tiling_on_sc=False),
)
def _sc_spmm(src_hbm, dst_hbm, w_hbm, x_hbm, out0_hbm, out1_hbm,
             srcb, dstb, wb, rowsq, sc0, sc1, acc_sh,
             gsem0, gsem1, ssem0, ssem1, isem):
    cid = lax.axis_index("c")
    sid = lax.axis_index("s")
    wid = sid * _NC + cid
    last = sid == _NS - 1
    gsem = (gsem0, gsem1)
    ssem = (ssem0, ssem1)
    scb = (sc0, sc1)
    nfull = jnp.where(last, 6, 10)

    def issue_idx(blk, buf):
        c0 = blk * _JB
        pltpu.async_copy(src_hbm.at[wid, pl.ds(c0, _JB)], srcb.at[buf], isem)
        pltpu.async_copy(dst_hbm.at[wid, pl.ds(c0 * 2, _JB * 2)],
                         dstb.at[buf], isem)
        pltpu.async_copy(
            w_hbm.at[pl.ds((wid * _CPW + c0) * _C, _JB * _C)],
            wb.at[pl.ds(buf * (_JB * _C), _JB * _C)], isem)

    def wait_idx(buf):
        pltpu.make_async_copy(
            src_hbm.at[wid, pl.ds(0, _JB)], srcb.at[buf], isem).wait()
        pltpu.make_async_copy(
            dst_hbm.at[wid, pl.ds(0, _JB * 2)], dstb.at[buf], isem).wait()
        pltpu.make_async_copy(
            w_hbm.at[pl.ds(0, _JB * _C)],
            wb.at[pl.ds(buf * (_JB * _C), _JB * _C)], isem).wait()

    def issue_gather(buf, ibuf, k):
        pltpu.async_copy(x_hbm.at[srcb.at[ibuf, k]], rowsq.at[buf], gsem[buf])

    def wait_gather(buf):
        pltpu.make_async_copy(
            x_hbm.at[srcb.at[0, 0]], rowsq.at[buf], gsem[buf]).wait()

    def issue_scatter(h, ibuf, k):
        pltpu.async_copy(scb[h], acc_sh.at[dstb.at[ibuf, 2 * k + h]],
                         ssem[h], add=True)

    def wait_scatter(h):
        pltpu.make_async_copy(scb[h], acc_sh.at[dstb.at[0, 0]],
                              ssem[h]).wait()

    def scale_half(buf, ibuf, k, h):
        @pl.loop(0, _HC, unroll=4)
        def _(j):
            wv = plsc.load_gather(
                wb, [jnp.full((16,), ibuf * (_JB * _C) + k * _C + h * _HC,
                              jnp.int32) + j])
            for g in range(4):
                vi = rowsq[buf, h * _HC + j, pl.ds(g * 16, 16)]
                ve = plsc.bitcast(vi << 16, jnp.float32)
                vo = plsc.bitcast(vi & jnp.int32(-65536), jnp.float32)
                scb[h][j, pl.ds(g * 32, 16)] = ve * wv
                scb[h][j, pl.ds(g * 32 + 16, 16)] = vo * wv

    issue_idx(0, 0)

    @pl.loop(0, _HC)
    def _(i):
        for c8 in range(8):
            sc0[i, pl.ds(c8 * 16, 16)] = jnp.zeros((16,), jnp.float32)

    @pl.loop(0, nfull)
    def _(k):
        pltpu.sync_copy(sc0, acc_sh.at[pl.ds(sid * _RPT + k * _HC, _HC)])

    @pl.when(last)
    def _():
        pltpu.sync_copy(sc0.at[pl.ds(0, 16)], acc_sh.at[pl.ds(9984, 16)])

    plsc.subcore_barrier()

    wait_idx(0)
    issue_gather(0, 0, 0)

    @pl.loop(0, _NBLK // 2)
    def _(jj2):
        for half in (0, 1):
            blk = jj2 * 2 + half
            for k in range(_JB):
                p = k % 2
                wait_gather(p)
                if k < _JB - 1:
                    issue_gather(1 - p, half, k + 1)
                if k == _JB - 1:
                    if half == 0:
                        wait_idx(1)
                        issue_gather(0, 1, 0)
                    else:
                        @pl.when(jj2 < _NBLK // 2 - 1)
                        def _():
                            wait_idx(0)
                            issue_gather(0, 0, 0)
                if half == 0 and k == 0:
                    @pl.when(jj2 > 0)
                    def _():
                        wait_scatter(0)
                else:
                    wait_scatter(0)
                scale_half(p, half, k, 0)
                issue_scatter(0, half, k)
                if half == 0 and k == 0:
                    @pl.when(jj2 > 0)
                    def _():
                        wait_scatter(1)
                else:
                    wait_scatter(1)
                if k == 0:
                    if half == 0:
                        issue_idx(blk + 1, 1)
                    else:
                        @pl.when(jj2 < _NBLK // 2 - 1)
                        def _():
                            issue_idx(blk + 1, 0)
                scale_half(p, half, k, 1)
                issue_scatter(1, half, k)

    wait_scatter(0)
    wait_scatter(1)
    plsc.subcore_barrier()

    for c, out_hbm in ((0, out0_hbm), (1, out1_hbm)):
        @pl.when(cid == c)
        def _():
            @pl.loop(0, nfull)
            def _(k):
                base = sid * _RPT + k * _HC
                pltpu.sync_copy(acc_sh.at[pl.ds(base, _HC)], sc0)
                pltpu.sync_copy(sc0, out_hbm.at[pl.ds(base, _HC)])

            @pl.when(last)
            def _():
                pltpu.sync_copy(acc_sh.at[pl.ds(9984, 16)],
                                sc0.at[pl.ds(0, 16)])
                pltpu.sync_copy(sc0.at[pl.ds(0, 16)],
                                out_hbm.at[pl.ds(9984, 16)])


@functools.partial(
    pl.kernel,
    out_type=jax.ShapeDtypeStruct((_GP, DQ), jnp.int32),
    mesh=_SC_MESH,
    scratch_types=[
        pltpu.VMEM((_GW, _C), jnp.int32),
        pltpu.VMEM((2, _C, DQ), jnp.int32),
        pltpu.SemaphoreType.DMA,
        pltpu.SemaphoreType.DMA,
        pltpu.SemaphoreType.DMA,
        pltpu.SemaphoreType.DMA,
    ],
    compiler_params=pltpu.CompilerParams(needs_layout_passes=False,
                                         use_tc_tiling_on_sc=False),
)
def _sc_gather(ids_hbm, x_hbm, out_hbm, ids_v, rows, gsem0, gsem1,
               osem0, osem1):
    cid = lax.axis_index("c")
    sid = lax.axis_index("s")
    wid = sid * _NC + cid
    gsem = (gsem0, gsem1)
    osem = (osem0, osem1)

    pltpu.sync_copy(ids_hbm.at[wid], ids_v)

    def issue_g(buf, k):
        pltpu.async_copy(x_hbm.at[ids_v.at[k]], rows.at[buf], gsem[buf])

    def wait_g(buf):
        pltpu.make_async_copy(
            x_hbm.at[ids_v.at[0]], rows.at[buf], gsem[buf]).wait()

    def issue_o(buf, k):
        pltpu.async_copy(
            rows.at[buf], out_hbm.at[pl.ds((wid * _GW + k) * _C, _C)],
            osem[buf])

    def wait_o(buf):
        pltpu.make_async_copy(
            rows.at[buf], out_hbm.at[pl.ds(0, _C)], osem[buf]).wait()

    issue_g(0, 0)
    for k in range(_GW):
        p = k % 2
        wait_g(p)
        if k + 1 < _GW:
            if k >= 1:
                wait_o(1 - p)
            issue_g(1 - p, k + 1)
        issue_o(p, k)
    wait_o(0)
    wait_o(1)


def _pack_body(x_ref, o_ref):
    o_ref[...] = x_ref[...].astype(jnp.bfloat16)


def _tc_pack(x):
    blk = 2000
    return pl.pallas_call(
        _pack_body,
        grid=(N_ITEMS // blk,),
        in_specs=[pl.BlockSpec((blk, D), lambda i: (i, 0))],
        out_specs=pl.BlockSpec((blk, D), lambda i: (i, 0)),
        out_shape=jax.ShapeDtypeStruct((N_ITEMS, D), jnp.bfloat16),
    )(x)


def _mm_body(relu, p0_ref, p1_ref, w_ref, b_ref, o_ref):
    x = p0_ref[...] + p1_ref[...]
    y = jnp.dot(x, w_ref[...], preferred_element_type=jnp.float32) + b_ref[...]
    if relu:
        y = jnp.maximum(y, 0.0)
    o_ref[...] = y.astype(jnp.bfloat16)


def _tc_mm(p0, p1, w, b, relu):
    blk = 2000
    return pl.pallas_call(
        functools.partial(_mm_body, relu),
        grid=(N_ITEMS // blk,),
        in_specs=[
            pl.BlockSpec((blk, D), lambda i: (i, 0)),
            pl.BlockSpec((blk, D), lambda i: (i, 0)),
            pl.BlockSpec((D, D), lambda i: (0, 0)),
            pl.BlockSpec((1, D), lambda i: (0, 0)),
        ],
        out_specs=pl.BlockSpec((blk, D), lambda i: (i, 0)),
        out_shape=jax.ShapeDtypeStruct((N_ITEMS, D), jnp.bfloat16),
    )(p0, p1, w, b)


def _gru_body(first, seq_ref, wih_ref, whh_ref, bih_ref, bhh_ref, *rest):
    if first:
        o_ref, h_s = rest
    else:
        h0_ref, o_ref, h_s = rest
    t = pl.program_id(0)

    @pl.when(t == 0)
    def _():
        if first:
            h_s[...] = jnp.zeros_like(h_s)
        else:
            h_s[...] = h0_ref[...]

    x = seq_ref[...]
    h = h_s[...]
    nt = (((1,), (1,)), ((), ()))
    gi = lax.dot_general(x, wih_ref[...], nt,
                         preferred_element_type=jnp.float32) + bih_ref[...]
    gh = lax.dot_general(h.astype(jnp.bfloat16), whh_ref[...], nt,
                         preferred_element_type=jnp.float32) + bhh_ref[...]
    r = jax.nn.sigmoid(gi[:, :H] + gh[:, :H])
    z = jax.nn.sigmoid(gi[:, H:2 * H] + gh[:, H:2 * H])
    n = jnp.tanh(gi[:, 2 * H:] + r * gh[:, 2 * H:])
    hn = (1.0 - z) * n + z * h
    h_s[...] = hn

    @pl.when(t == _TH - 1)
    def _():
        o_ref[...] = hn


def _tc_gru(seq, w_ih, w_hh, b_ih, b_hh, h0=None):
    first = h0 is None
    in_specs = [
        pl.BlockSpec((B, D), lambda t: (t, 0)),
        pl.BlockSpec((3 * H, D), lambda t: (0, 0)),
        pl.BlockSpec((3 * H, H), lambda t: (0, 0)),
        pl.BlockSpec((1, 3 * H), lambda t: (0, 0)),
        pl.BlockSpec((1, 3 * H), lambda t: (0, 0)),
    ]
    args = [seq, w_ih, w_hh, b_ih, b_hh]
    if not first:
        in_specs.append(pl.BlockSpec((B, H), lambda t: (0, 0)))
        args.append(h0)
    return pl.pallas_call(
        functools.partial(_gru_body, first),
        grid=(_TH,),
        in_specs=in_specs,
        out_specs=pl.BlockSpec((B, H), lambda t: (0, 0)),
        out_shape=jax.ShapeDtypeStruct((B, H), jnp.float32),
        scratch_shapes=[pltpu.VMEM((B, H), jnp.float32)],
    )(*args)


def _fc_body(h_ref, w_ref, b_ref, o_ref):
    nt = (((1,), (1,)), ((), ()))
    o_ref[...] = lax.dot_general(h_ref[...], w_ref[...], nt,
                                 preferred_element_type=jnp.float32) + b_ref[...]


def _tc_fc(h, fcw, fcb):
    vblk = 2048
    return pl.pallas_call(
        _fc_body,
        grid=(pl.cdiv(N_ITEMS, vblk),),
        in_specs=[
            pl.BlockSpec((B, H), lambda j: (0, 0)),
            pl.BlockSpec((vblk, H), lambda j: (j, 0)),
            pl.BlockSpec((1, vblk), lambda j: (0, j)),
        ],
        out_specs=pl.BlockSpec((B, vblk), lambda j: (0, j)),
        out_shape=jax.ShapeDtypeStruct((B, N_ITEMS), jnp.float32),
    )(h, fcw, fcb)


def kernel(item_sequence, edge_index, edge_weight, emb, W1, b1, W2, b2,
           W_ih, W_hh, b_ih, b_hh, fc_W, fc_b):
    src = edge_index[0].astype(jnp.int32)
    dst = edge_index[1].astype(jnp.int32)
    w = edge_weight.astype(jnp.float32)
    pad = _EP - E
    srcp = jnp.pad(src, (0, pad)).reshape(_NW, _CPW, _C)
    dstp = jnp.pad(dst, (0, pad)).reshape(_NW, _CPW * 2, _HC)
    wp = jnp.pad(w, (0, pad))

    perm = jnp.asarray(_PERM)
    p0, p1 = _sc_spmm(srcp, dstp, wp, _as_i32_pairs(_tc_pack(emb)))
    x1b = _tc_mm(p0, p1, W1[perm], b1.reshape(1, D), relu=True)
    q0, q1 = _sc_spmm(srcp, dstp, wp, _as_i32_pairs(x1b))
    x2b = _tc_mm(q0, q1, W2[perm], b2.reshape(1, D), relu=False)

    x2q = _as_i32_pairs(x2b)
    iseq = item_sequence.astype(jnp.int32)
    wihb = W_ih.astype(jnp.bfloat16)
    whhb = W_hh.astype(jnp.bfloat16)
    bihr = b_ih.reshape(1, 3 * H)
    bhhr = b_hh.reshape(1, 3 * H)

    def gather_half(ids_half):
        ids = ids_half.T.reshape(_TH * B)
        idsp = jnp.pad(ids, (0, _GP - _TH * B)).reshape(_NW, _GW, _C)
        sq = _sc_gather(idsp, x2q)
        return jax.lax.bitcast_convert_type(sq, jnp.bfloat16).reshape(_GP, D)

    seq_a = gather_half(iseq[:, :_TH])
    seq_b = gather_half(iseq[:, _TH:])
    h1 = _tc_gru(seq_a, wihb, whhb, bihr, bhhr)
    h = _tc_gru(seq_b, wihb, whhb, bihr, bhhr, h0=h1)

    return _tc_fc(h, fc_W, fc_b.reshape(1, N_ITEMS))


# speedup vs baseline: 1.1406x; 1.1406x over previous
import functools

import numpy as np
import jax
import jax.numpy as jnp
from jax import lax
from jax.experimental import pallas as pl
from jax.experimental.pallas import tpu as pltpu
from jax.experimental.pallas import tpu_sc as plsc

N_ITEMS = 10000
D = 128
DQ = D // 2
H = 256
E = 320000
B = 1024
T = 50

_NC = 2
_NS = 16
_NW = _NC * _NS
_C = 128
_HC = _C // 2
_CPW = 80
_JB = 8
_NBLK = _CPW // _JB
_EP = _NW * _CPW * _C
_RPT = 640
_GW = 13
_GP = _NW * _GW * _C
_VP = 10240

_PERM = np.array(
    [32 * (p // 32) + (2 * (p % 32) if p % 32 < 16 else 2 * (p % 32 - 16) + 1)
     for p in range(D)], dtype=np.int32)

_SC_MESH = plsc.VectorSubcoreMesh(core_axis_name="c", subcore_axis_name="s")


def _as_i32_pairs(xb):
    return jax.lax.bitcast_convert_type(
        xb.reshape(xb.shape[0], DQ, 2), jnp.int32)


@functools.partial(
    pl.kernel,
    out_type=[jax.ShapeDtypeStruct((N_ITEMS, D), jnp.float32),
              jax.ShapeDtypeStruct((N_ITEMS, D), jnp.float32)],
    mesh=_SC_MESH,
    scratch_types=[
        pltpu.VMEM((2, _JB, _C), jnp.int32),
        pltpu.VMEM((2, _JB * 2, _HC), jnp.int32),
        pltpu.VMEM((2 * _JB * _C,), jnp.float32),
        pltpu.VMEM((2, _C, DQ), jnp.int32),
        pltpu.VMEM((_HC, D), jnp.float32),
        pltpu.VMEM((_HC, D), jnp.float32),
        pltpu.VMEM_SHARED((N_ITEMS, D), jnp.float32),
        pltpu.SemaphoreType.DMA,
        pltpu.SemaphoreType.DMA,
        pltpu.SemaphoreType.DMA,
        pltpu.SemaphoreType.DMA,
        pltpu.SemaphoreType.DMA,
    ],
    compiler_params=pltpu.CompilerParams(needs_layout_passes=False,
                                         use_tc_tiling_on_sc=False),
)
def _sc_spmm(src_hbm, dst_hbm, w_hbm, x_hbm, out0_hbm, out1_hbm,
             srcb, dstb, wb, rowsq, sc0, sc1, acc_sh,
             gsem0, gsem1, ssem0, ssem1, isem):
    cid = lax.axis_index("c")
    sid = lax.axis_index("s")
    wid = sid * _NC + cid
    last = sid == _NS - 1
    gsem = (gsem0, gsem1)
    ssem = (ssem0, ssem1)
    scb = (sc0, sc1)
    nfull = jnp.where(last, 6, 10)

    def issue_idx(blk, buf):
        c0 = blk * _JB
        pltpu.async_copy(src_hbm.at[wid, pl.ds(c0, _JB)], srcb.at[buf], isem)
        pltpu.async_copy(dst_hbm.at[wid, pl.ds(c0 * 2, _JB * 2)],
                         dstb.at[buf], isem)
        pltpu.async_copy(
            w_hbm.at[pl.ds((wid * _CPW + c0) * _C, _JB * _C)],
            wb.at[pl.ds(buf * (_JB * _C), _JB * _C)], isem)

    def wait_idx(buf):
        pltpu.make_async_copy(
            src_hbm.at[wid, pl.ds(0, _JB)], srcb.at[buf], isem).wait()
        pltpu.make_async_copy(
            dst_hbm.at[wid, pl.ds(0, _JB * 2)], dstb.at[buf], isem).wait()
        pltpu.make_async_copy(
            w_hbm.at[pl.ds(0, _JB * _C)],
            wb.at[pl.ds(buf * (_JB * _C), _JB * _C)], isem).wait()

    def issue_gather(buf, ibuf, k):
        pltpu.async_copy(x_hbm.at[srcb.at[ibuf, k]], rowsq.at[buf], gsem[buf])

    def wait_gather(buf):
        pltpu.make_async_copy(
            x_hbm.at[srcb.at[0, 0]], rowsq.at[buf], gsem[buf]).wait()

    def issue_scatter(h, ibuf, k):
        pltpu.async_copy(scb[h], acc_sh.at[dstb.at[ibuf, 2 * k + h]],
                         ssem[h], add=True)

    def wait_scatter(h):
        pltpu.make_async_copy(scb[h], acc_sh.at[dstb.at[0, 0]],
                              ssem[h]).wait()

    def scale_half(buf, ibuf, k, h):
        @pl.loop(0, _HC, unroll=4)
        def _(j):
            wv = plsc.load_gather(
                wb, [jnp.full((16,), ibuf * (_JB * _C) + k * _C + h * _HC,
                              jnp.int32) + j])
            for g in range(4):
                vi = rowsq[buf, h * _HC + j, pl.ds(g * 16, 16)]
                ve = plsc.bitcast(vi << 16, jnp.float32)
                vo = plsc.bitcast(vi & jnp.int32(-65536), jnp.float32)
                scb[h][j, pl.ds(g * 32, 16)] = ve * wv
                scb[h][j, pl.ds(g * 32 + 16, 16)] = vo * wv

    issue_idx(0, 0)

    @pl.loop(0, _HC)
    def _(i):
        for c8 in range(8):
            sc0[i, pl.ds(c8 * 16, 16)] = jnp.zeros((16,), jnp.float32)

    @pl.loop(0, nfull)
    def _(k):
        pltpu.sync_copy(sc0, acc_sh.at[pl.ds(sid * _RPT + k * _HC, _HC)])

    @pl.when(last)
    def _():
        pltpu.sync_copy(sc0.at[pl.ds(0, 16)], acc_sh.at[pl.ds(9984, 16)])

    plsc.subcore_barrier()

    wait_idx(0)
    issue_gather(0, 0, 0)

    @pl.loop(0, _NBLK // 2)
    def _(jj2):
        for half in (0, 1):
            blk = jj2 * 2 + half
            for k in range(_JB):
                p = k % 2
                wait_gather(p)
                if k < _JB - 1:
                    issue_gather(1 - p, half, k + 1)
                if k == _JB - 1:
                    if half == 0:
                        wait_idx(1)
                        issue_gather(0, 1, 0)
                    else:
                        @pl.when(jj2 < _NBLK // 2 - 1)
                        def _():
                            wait_idx(0)
                            issue_gather(0, 0, 0)
                if half == 0 and k == 0:
                    @pl.when(jj2 > 0)
                    def _():
                        wait_scatter(0)
                else:
                    wait_scatter(0)
                scale_half(p, half, k, 0)
                issue_scatter(0, half, k)
                if half == 0 and k == 0:
                    @pl.when(jj2 > 0)
                    def _():
                        wait_scatter(1)
                else:
                    wait_scatter(1)
                if k == 0:
                    if half == 0:
                        issue_idx(blk + 1, 1)
                    else:
                        @pl.when(jj2 < _NBLK // 2 - 1)
                        def _():
                            issue_idx(blk + 1, 0)
                scale_half(p, half, k, 1)
                issue_scatter(1, half, k)

    wait_scatter(0)
    wait_scatter(1)
    plsc.subcore_barrier()

    for c, out_hbm in ((0, out0_hbm), (1, out1_hbm)):
        @pl.when(cid == c)
        def _():
            @pl.loop(0, nfull)
            def _(k):
                base = sid * _RPT + k * _HC
                pltpu.sync_copy(acc_sh.at[pl.ds(base, _HC)], sc0)
                pltpu.sync_copy(sc0, out_hbm.at[pl.ds(base, _HC)])

            @pl.when(last)
            def _():
                pltpu.sync_copy(acc_sh.at[pl.ds(9984, 16)],
                                sc0.at[pl.ds(0, 16)])
                pltpu.sync_copy(sc0.at[pl.ds(0, 16)],
                                out_hbm.at[pl.ds(9984, 16)])


@functools.partial(
    pl.kernel,
    out_type=jax.ShapeDtypeStruct((_GP, DQ), jnp.int32),
    mesh=_SC_MESH,
    scratch_types=[
        pltpu.VMEM((_GW, _C), jnp.int32),
        pltpu.VMEM((2, _C, DQ), jnp.int32),
        pltpu.SemaphoreType.DMA,
        pltpu.SemaphoreType.DMA,
        pltpu.SemaphoreType.DMA,
        pltpu.SemaphoreType.DMA,
    ],
    compiler_params=pltpu.CompilerParams(needs_layout_passes=False,
                                         use_tc_tiling_on_sc=False),
)
def _sc_gather(ids_hbm, x_hbm, out_hbm, ids_v, rows, gsem0, gsem1,
               osem0, osem1):
    cid = lax.axis_index("c")
    sid = lax.axis_index("s")
    wid = sid * _NC + cid
    gsem = (gsem0, gsem1)
    osem = (osem0, osem1)

    pltpu.sync_copy(ids_hbm.at[wid], ids_v)

    def issue_g(buf, k):
        pltpu.async_copy(x_hbm.at[ids_v.at[k]], rows.at[buf], gsem[buf])

    def wait_g(buf):
        pltpu.make_async_copy(
            x_hbm.at[ids_v.at[0]], rows.at[buf], gsem[buf]).wait()

    def issue_o(buf, k):
        pltpu.async_copy(
            rows.at[buf], out_hbm.at[pl.ds((wid * _GW + k) * _C, _C)],
            osem[buf])

    def wait_o(buf):
        pltpu.make_async_copy(
            rows.at[buf], out_hbm.at[pl.ds(0, _C)], osem[buf]).wait()

    issue_g(0, 0)
    for k in range(_GW):
        p = k % 2
        wait_g(p)
        if k + 1 < _GW:
            if k >= 1:
                wait_o(1 - p)
            issue_g(1 - p, k + 1)
        issue_o(p, k)
    wait_o(0)
    wait_o(1)


def _pack_body(x_ref, o_ref):
    o_ref[...] = x_ref[...].astype(jnp.bfloat16)


def _tc_pack(x):
    blk = 2000
    return pl.pallas_call(
        _pack_body,
        grid=(N_ITEMS // blk,),
        in_specs=[pl.BlockSpec((blk, D), lambda i: (i, 0))],
        out_specs=pl.BlockSpec((blk, D), lambda i: (i, 0)),
        out_shape=jax.ShapeDtypeStruct((N_ITEMS, D), jnp.bfloat16),
    )(x)


def _mm_body(relu, p0_ref, p1_ref, w_ref, b_ref, o_ref):
    x = p0_ref[...] + p1_ref[...]
    y = jnp.dot(x, w_ref[...], preferred_element_type=jnp.float32) + b_ref[...]
    if relu:
        y = jnp.maximum(y, 0.0)
    o_ref[...] = y.astype(jnp.bfloat16)


def _tc_mm(p0, p1, w, b, relu):
    blk = 2000
    return pl.pallas_call(
        functools.partial(_mm_body, relu),
        grid=(N_ITEMS // blk,),
        in_specs=[
            pl.BlockSpec((blk, D), lambda i: (i, 0)),
            pl.BlockSpec((blk, D), lambda i: (i, 0)),
            pl.BlockSpec((D, D), lambda i: (0, 0)),
            pl.BlockSpec((1, D), lambda i: (0, 0)),
        ],
        out_specs=pl.BlockSpec((blk, D), lambda i: (i, 0)),
        out_shape=jax.ShapeDtypeStruct((N_ITEMS, D), jnp.bfloat16),
    )(p0, p1, w, b)


def _gru_body(seq_ref, wih_ref, whh_ref, bih_ref, bhh_ref, o_ref, h_s):
    t = pl.program_id(0)

    @pl.when(t == 0)
    def _():
        h_s[...] = jnp.zeros_like(h_s)

    x = seq_ref[...]
    h = h_s[...]
    nt = (((1,), (1,)), ((), ()))
    gi = lax.dot_general(x, wih_ref[...], nt,
                         preferred_element_type=jnp.float32) + bih_ref[...]
    gh = lax.dot_general(h.astype(jnp.bfloat16), whh_ref[...], nt,
                         preferred_element_type=jnp.float32) + bhh_ref[...]
    r = jax.nn.sigmoid(gi[:, :H] + gh[:, :H])
    z = jax.nn.sigmoid(gi[:, H:2 * H] + gh[:, H:2 * H])
    n = jnp.tanh(gi[:, 2 * H:] + r * gh[:, 2 * H:])
    hn = (1.0 - z) * n + z * h
    h_s[...] = hn

    @pl.when(t == T - 1)
    def _():
        o_ref[...] = hn


def _tc_gru(seq, w_ih, w_hh, b_ih, b_hh):
    return pl.pallas_call(
        _gru_body,
        grid=(T,),
        in_specs=[
            pl.BlockSpec((B, D), lambda t: (t, 0)),
            pl.BlockSpec((3 * H, D), lambda t: (0, 0)),
            pl.BlockSpec((3 * H, H), lambda t: (0, 0)),
            pl.BlockSpec((1, 3 * H), lambda t: (0, 0)),
            pl.BlockSpec((1, 3 * H), lambda t: (0, 0)),
        ],
        out_specs=pl.BlockSpec((B, H), lambda t: (0, 0)),
        out_shape=jax.ShapeDtypeStruct((B, H), jnp.float32),
        scratch_shapes=[pltpu.VMEM((B, H), jnp.float32)],
    )(seq, w_ih, w_hh, b_ih, b_hh)


def _fc_body(h_ref, w_ref, b_ref, o_ref):
    nt = (((1,), (1,)), ((), ()))
    o_ref[...] = lax.dot_general(h_ref[...], w_ref[...], nt,
                                 preferred_element_type=jnp.float32) + b_ref[...]


def _tc_fc(h, fcw, fcb):
    vblk = 2048
    return pl.pallas_call(
        _fc_body,
        grid=(pl.cdiv(N_ITEMS, vblk),),
        in_specs=[
            pl.BlockSpec((B, H), lambda j: (0, 0)),
            pl.BlockSpec((vblk, H), lambda j: (j, 0)),
            pl.BlockSpec((1, vblk), lambda j: (0, j)),
        ],
        out_specs=pl.BlockSpec((B, vblk), lambda j: (0, j)),
        out_shape=jax.ShapeDtypeStruct((B, N_ITEMS), jnp.float32),
    )(h, fcw, fcb)


def kernel(item_sequence, edge_index, edge_weight, emb, W1, b1, W2, b2,
           W_ih, W_hh, b_ih, b_hh, fc_W, fc_b):
    src = edge_index[0].astype(jnp.int32)
    dst = edge_index[1].astype(jnp.int32)
    w = edge_weight.astype(jnp.float32)
    pad = _EP - E
    srcp = jnp.pad(src, (0, pad)).reshape(_NW, _CPW, _C)
    dstp = jnp.pad(dst, (0, pad)).reshape(_NW, _CPW * 2, _HC)
    wp = jnp.pad(w, (0, pad))

    perm = jnp.asarray(_PERM)
    p0, p1 = _sc_spmm(srcp, dstp, wp, _as_i32_pairs(_tc_pack(emb)))
    x1b = _tc_mm(p0, p1, W1[perm], b1.reshape(1, D), relu=True)
    q0, q1 = _sc_spmm(srcp, dstp, wp, _as_i32_pairs(x1b))
    x2b = _tc_mm(q0, q1, W2[perm], b2.reshape(1, D), relu=False)

    ids = item_sequence.astype(jnp.int32).T.reshape(T * B)
    idsp = jnp.pad(ids, (0, _GP - T * B)).reshape(_NW, _GW, _C)
    seqq = _sc_gather(idsp, _as_i32_pairs(x2b))
    seq = jax.lax.bitcast_convert_type(seqq, jnp.bfloat16).reshape(_GP, D)

    h = _tc_gru(seq, W_ih.astype(jnp.bfloat16), W_hh.astype(jnp.bfloat16),
                b_ih.reshape(1, 3 * H), b_hh.reshape(1, 3 * H))

    return _tc_fc(h, fc_W, fc_b.reshape(1, N_ITEMS))
